# baseline (speedup 1.0000x reference)
import functools

import jax
import jax.numpy as jnp
from jax import lax
from jax.experimental import pallas as pl
from jax.experimental.pallas import tpu as pltpu
from jax.experimental.pallas import tpu_sc as plsc

_B = 16384
_C = 1000
_NW = 32
_RPW = _B // _NW
_EPW = _RPW * _C
_CH = 32000
_NCH = _EPW // _CH


def _sc_body(p_hbm, y_hbm, out_hbm, buf, yv, idxv, gv, accv, sd0, sd1, sg):
    w = lax.axis_index("s") * 2 + lax.axis_index("c")
    base_e = w * _EPW

    sems = (sd0, sd1)

    def copy(t, b):
        return pltpu.make_async_copy(
            p_hbm.at[pl.ds(base_e + t * _CH, _CH)], buf.at[b], sems[b]
        )

    copy(0, 0).start()

    pltpu.sync_copy(y_hbm.at[pl.ds(w * _RPW, _RPW)], yv)
    lane = lax.broadcasted_iota(jnp.int32, (16,), 0)
    for j in range(_RPW // 16):
        rvec = lane + (w * _RPW + j * 16)
        idxv[j // 8, pl.ds((j % 8) * 16, 16)] = rvec * _C + yv[pl.ds(j * 16, 16)]
    for k in range(_RPW // 128):
        pltpu.async_copy(p_hbm.at[idxv.at[k]], gv.at[k], sg).wait()
    gacc = jnp.zeros((16,), jnp.float32)
    for k in range(_RPW // 128):
        for j in range(8):
            gacc = gacc + gv[k, pl.ds(j * 16, 16)]

    acc = jnp.zeros((16,), jnp.float32)
    for t in range(_NCH):
        b = t % 2
        if t + 1 < _NCH:
            copy(t + 1, 1 - b).start()
        copy(t, b).wait()

        def ibody(j, a):
            v = buf[b, pl.ds(j * 16, 16)]
            return a + v * v

        acc = lax.fori_loop(0, _CH // 16, ibody, acc, unroll=8)

    accv[pl.ds(0, 16)] = acc
    accv[pl.ds(16, 16)] = gacc
    pltpu.sync_copy(accv, out_hbm.at[w])


def _sc_partials(p_flat, y32):
    mesh = plsc.VectorSubcoreMesh(core_axis_name="c", subcore_axis_name="s")
    run = functools.partial(
        pl.kernel,
        mesh=mesh,
        out_type=jax.ShapeDtypeStruct((_NW, 32), jnp.float32),
        scratch_types=[
            pltpu.VMEM((2, _CH), jnp.float32),
            pltpu.VMEM((_RPW,), jnp.int32),
            pltpu.VMEM((_RPW // 128, 128), jnp.int32),
            pltpu.VMEM((_RPW // 128, 128), jnp.float32),
            pltpu.VMEM((32,), jnp.float32),
            pltpu.SemaphoreType.DMA,
            pltpu.SemaphoreType.DMA,
            pltpu.SemaphoreType.DMA,
        ],
    )(_sc_body)
    return run(p_flat, y32)


def _combine_body(sc_ref, out_ref):
    sc = sc_ref[...]
    total = jnp.sum(sc[:, 0:16]) - 2.0 * jnp.sum(sc[:, 16:32])
    out_ref[0, 0] = (total + jnp.float32(_B)) / jnp.float32(_B)


def kernel(probs, y):
    y32 = y.astype(jnp.int32)
    partials = _sc_partials(probs.reshape(-1), y32)
    out = pl.pallas_call(
        _combine_body,
        in_specs=[pl.BlockSpec(memory_space=pltpu.VMEM)],
        out_specs=pl.BlockSpec(memory_space=pltpu.SMEM),
        out_shape=jax.ShapeDtypeStruct((1, 1), jnp.float32),
    )(partials)
    return out[0, 0]


# speedup vs baseline: 1.0248x; 1.0248x over previous
import functools

import jax
import jax.numpy as jnp
from jax import lax
from jax.experimental import pallas as pl
from jax.experimental.pallas import tpu as pltpu
from jax.experimental.pallas import tpu_sc as plsc

_B = 16384
_C = 1000
_NW = 32
_RPW = _B // _NW
_EPW = _RPW * _C
_CH = 32000
_NCH = _EPW // _CH


def _sc_body(p_hbm, y_hbm, out_hbm, buf, yv, idxv, gv, accv, sd0, sd1, sg):
    w = lax.axis_index("s") * 2 + lax.axis_index("c")
    base_e = w * _EPW

    sems = (sd0, sd1)

    def copy(t, b):
        return pltpu.make_async_copy(
            p_hbm.at[pl.ds(base_e + t * _CH, _CH)], buf.at[b], sems[b]
        )

    copy(0, 0).start()

    pltpu.sync_copy(y_hbm.at[pl.ds(w * _RPW, _RPW)], yv)
    lane = lax.broadcasted_iota(jnp.int32, (16,), 0)
    for j in range(_RPW // 16):
        rvec = lane + (w * _RPW + j * 16)
        idxv[j // 8, pl.ds((j % 8) * 16, 16)] = rvec * _C + yv[pl.ds(j * 16, 16)]
    for k in range(_RPW // 128):
        pltpu.async_copy(p_hbm.at[idxv.at[k]], gv.at[k], sg).wait()
    gacc = jnp.zeros((16,), jnp.float32)
    for k in range(_RPW // 128):
        for j in range(8):
            gacc = gacc + gv[k, pl.ds(j * 16, 16)]

    accs = tuple(jnp.zeros((16,), jnp.float32) for _ in range(4))
    for t in range(_NCH):
        b = t % 2
        if t + 1 < _NCH:
            copy(t + 1, 1 - b).start()
        copy(t, b).wait()

        def ibody(e, c, _b=b):
            a0, a1, a2, a3 = c
            v0 = buf[_b, pl.ds(e, 16)]
            v1 = buf[_b, pl.ds(e + 16, 16)]
            v2 = buf[_b, pl.ds(e + 32, 16)]
            v3 = buf[_b, pl.ds(e + 48, 16)]
            return (a0 + v0 * v0, a1 + v1 * v1, a2 + v2 * v2, a3 + v3 * v3)

        accs = plsc.parallel_loop(0, _CH, step=64, unroll=8, carry=accs)(ibody)

    acc = (accs[0] + accs[1]) + (accs[2] + accs[3])
    accv[pl.ds(0, 16)] = acc
    accv[pl.ds(16, 16)] = gacc
    pltpu.sync_copy(accv, out_hbm.at[w])


def _sc_partials(p_flat, y32):
    mesh = plsc.VectorSubcoreMesh(core_axis_name="c", subcore_axis_name="s")
    run = functools.partial(
        pl.kernel,
        mesh=mesh,
        out_type=jax.ShapeDtypeStruct((_NW, 32), jnp.float32),
        scratch_types=[
            pltpu.VMEM((2, _CH), jnp.float32),
            pltpu.VMEM((_RPW,), jnp.int32),
            pltpu.VMEM((_RPW // 128, 128), jnp.int32),
            pltpu.VMEM((_RPW // 128, 128), jnp.float32),
            pltpu.VMEM((32,), jnp.float32),
            pltpu.SemaphoreType.DMA,
            pltpu.SemaphoreType.DMA,
            pltpu.SemaphoreType.DMA,
        ],
    )(_sc_body)
    return run(p_flat, y32)


def _combine_body(sc_ref, out_ref):
    sc = sc_ref[...]
    total = jnp.sum(sc[:, 0:16]) - 2.0 * jnp.sum(sc[:, 16:32])
    out_ref[0, 0] = (total + jnp.float32(_B)) / jnp.float32(_B)


def kernel(probs, y):
    y32 = y.astype(jnp.int32)
    partials = _sc_partials(probs.reshape(-1), y32)
    out = pl.pallas_call(
        _combine_body,
        in_specs=[pl.BlockSpec(memory_space=pltpu.VMEM)],
        out_specs=pl.BlockSpec(memory_space=pltpu.SMEM),
        out_shape=jax.ShapeDtypeStruct((1, 1), jnp.float32),
    )(partials)
    return out[0, 0]


# speedup vs baseline: 1.2782x; 1.2473x over previous
import functools

import jax
import jax.numpy as jnp
from jax import lax
from jax.experimental import pallas as pl
from jax.experimental.pallas import tpu as pltpu
from jax.experimental.pallas import tpu_sc as plsc

_B = 16384
_C = 1000
_S = 8192

_NW = 32
_RPW = (_B - _S) // _NW
_RB = 32
_NCH = _RPW // _RB

_BR = 1024
_NBUF = 4
_NCHUNK = _S // _BR


def _sc_body(p_hbm, y_hbm, out_hbm, buf, yv, accv, sd0, sd1):
    w = lax.axis_index("s") * 2 + lax.axis_index("c")
    row0 = _S + w * _RPW
    sems = (sd0, sd1)

    def copy(t, b):
        return pltpu.make_async_copy(
            p_hbm.at[pl.ds(row0 + t * _RB, _RB), :], buf.at[b], sems[b]
        )

    copy(0, 0).start()
    pltpu.sync_copy(y_hbm.at[pl.ds(row0, _RPW)], yv)

    lane = lax.broadcasted_iota(jnp.int32, (16,), 0)
    zeros16 = jnp.zeros((16,), jnp.float32)

    acc = zeros16
    gacc = zeros16
    for t in range(_NCH):
        b = t % 2
        if t + 1 < _NCH:
            copy(t + 1, 1 - b).start()
        copy(t, b).wait()
        for r in range(_RB):
            lr = t * _RB + r
            ybc = lax.gather(
                yv[pl.ds((lr // 16) * 16, 16)],
                jnp.full((16, 1), lr % 16, jnp.int32),
                lax.GatherDimensionNumbers(
                    offset_dims=(),
                    collapsed_slice_dims=(0,),
                    start_index_map=(0,),
                ),
                slice_sizes=(1,),
                mode=lax.GatherScatterMode.PROMISE_IN_BOUNDS,
            )

            def ibody(e, c, _b=b, _r=r, _ybc=ybc):
                a, g = c
                v = buf[_b, _r, pl.ds(e, 16)]
                hit = (lane + e) == _ybc
                return a + v * v, g + jnp.where(hit, v, zeros16)

            acc, gacc = plsc.parallel_loop(
                0, 992, step=16, unroll=8, carry=(acc, gacc)
            )(ibody)
            vt = buf[b, r, pl.ds(984, 16)]
            keep = lane >= 8
            vt = jnp.where(keep, vt, zeros16)
            acc = acc + vt * vt
            gacc = gacc + jnp.where((lane + 984) == ybc, vt, zeros16)

    accv[pl.ds(0, 16)] = acc
    accv[pl.ds(16, 16)] = gacc
    pltpu.sync_copy(accv, out_hbm.at[w])


def _sc_partials(probs, y32):
    mesh = plsc.VectorSubcoreMesh(core_axis_name="c", subcore_axis_name="s")
    run = functools.partial(
        pl.kernel,
        mesh=mesh,
        out_type=jax.ShapeDtypeStruct((_NW, 32), jnp.float32),
        scratch_types=[
            pltpu.VMEM((2, _RB, _C), jnp.float32),
            pltpu.VMEM((_RPW,), jnp.int32),
            pltpu.VMEM((32,), jnp.float32),
            pltpu.SemaphoreType.DMA,
            pltpu.SemaphoreType.DMA,
        ],
    )(_sc_body)
    return run(probs, y32)


def _tc_body(y_ref, sc_ref, p_hbm, out_ref, b0, b1, b2, b3, s0, s1, s2, s3):
    bufs = (b0, b1, b2, b3)
    sems = (s0, s1, s2, s3)

    def copy(i, slot):
        return pltpu.make_async_copy(
            p_hbm.at[pl.ds(i * _BR, _BR), :], bufs[slot], sems[slot]
        )

    for s in range(_NBUF):
        copy(s, s).start()

    col = jax.lax.broadcasted_iota(jnp.int32, (_BR, _C), 1)
    acc = jnp.float32(0.0)
    for i in range(_NCHUNK):
        slot = i % _NBUF
        copy(i, slot).wait()
        p = bufs[slot][...]
        yb = y_ref[pl.ds(i * _BR, _BR), :]
        acc += jnp.sum(p * p) - 2.0 * jnp.sum(jnp.where(col == yb, p, 0.0))
        if i + _NBUF < _NCHUNK:
            copy(i + _NBUF, slot).start()

    sc = sc_ref[...]
    acc += jnp.sum(sc[:, 0:16]) - 2.0 * jnp.sum(sc[:, 16:32])
    out_ref[0, 0] = (acc + jnp.float32(_B)) / jnp.float32(_B)


def kernel(probs, y):
    y32 = y.astype(jnp.int32)
    partials = _sc_partials(probs, y32)
    out = pl.pallas_call(
        _tc_body,
        in_specs=[
            pl.BlockSpec(memory_space=pltpu.VMEM),
            pl.BlockSpec(memory_space=pltpu.VMEM),
            pl.BlockSpec(memory_space=pl.ANY),
        ],
        out_specs=pl.BlockSpec(memory_space=pltpu.SMEM),
        out_shape=jax.ShapeDtypeStruct((1, 1), jnp.float32),
        scratch_shapes=(
            [pltpu.VMEM((_BR, _C), jnp.float32) for _ in range(_NBUF)]
            + [pltpu.SemaphoreType.DMA for _ in range(_NBUF)]
        ),
    )(y32.reshape(_B, 1), partials, probs)
    return out[0, 0]


# speedup vs baseline: 1.5810x; 1.2369x over previous
import functools

import jax
import jax.numpy as jnp
from jax import lax
from jax.experimental import pallas as pl
from jax.experimental.pallas import tpu as pltpu
from jax.experimental.pallas import tpu_sc as plsc

_B = 16384
_C = 1000
_S = 10240

_NW = 32
_RPW = (_B - _S) // _NW
_RB = 32
_NCH = _RPW // _RB

_BR = 1024
_NBUF = 4
_NCHUNK = _S // _BR


def _sc_body(p_hbm, y_hbm, out_hbm, bufA, bufB, yv, accv, sd0, sd1):
    w = lax.axis_index("s") * 2 + lax.axis_index("c")
    row0 = _S + w * _RPW
    sems = (sd0, sd1)

    bufs = (bufA, bufB)

    def copy(t, b):
        return pltpu.make_async_copy(
            p_hbm.at[pl.ds(row0 + t * _RB, _RB), :], bufs[b], sems[b]
        )

    copy(0, 0).start()
    pltpu.sync_copy(y_hbm.at[pl.ds(row0, _RPW)], yv)

    lane = lax.broadcasted_iota(jnp.int32, (16,), 0)
    zeros16 = jnp.zeros((16,), jnp.float32)

    a0 = zeros16
    a1 = zeros16
    gacc = zeros16
    for t in range(_NCH):
        b = t % 2
        if t + 1 < _NCH:
            copy(t + 1, 1 - b).start()
        copy(t, b).wait()

        for r in range(_RB):
            lr = t * _RB + r
            ybc = lax.gather(
                yv[pl.ds((lr // 16) * 16, 16)],
                jnp.full((16, 1), lr % 16, jnp.int32),
                lax.GatherDimensionNumbers(
                    offset_dims=(),
                    collapsed_slice_dims=(0,),
                    start_index_map=(0,),
                ),
                slice_sizes=(1,),
                mode=lax.GatherScatterMode.PROMISE_IN_BOUNDS,
            )

            def ibody(e, c, _b=b, _r=r, _ybc=ybc):
                a, g = c
                v = bufs[_b][_r, pl.ds(e, 16)]
                hit = (lane + e) == _ybc
                return a + v * v, g + jnp.where(hit, v, zeros16)

            a0, gacc = plsc.parallel_loop(
                0, 992, step=16, unroll=8, carry=(a0, gacc)
            )(ibody)
            vt = bufs[b][r, pl.ds(984, 16)]
            vt = jnp.where(lane >= 8, vt, zeros16)
            a0 = a0 + vt * vt
            gacc = gacc + jnp.where((lane + 984) == ybc, vt, zeros16)

    accv[pl.ds(0, 16)] = a0 + a1
    accv[pl.ds(16, 16)] = gacc
    pltpu.sync_copy(accv, out_hbm.at[w])


def _sc_partials(probs, y32):
    mesh = plsc.VectorSubcoreMesh(core_axis_name="c", subcore_axis_name="s")
    run = functools.partial(
        pl.kernel,
        mesh=mesh,
        out_type=jax.ShapeDtypeStruct((_NW, 32), jnp.float32),
        scratch_types=[
            pltpu.VMEM((_RB, _C), jnp.float32),
            pltpu.VMEM((_RB, _C), jnp.float32),
            pltpu.VMEM((_RPW,), jnp.int32),
            pltpu.VMEM((32,), jnp.float32),
            pltpu.SemaphoreType.DMA,
            pltpu.SemaphoreType.DMA,
        ],
    )(_sc_body)
    return run(probs, y32)


def _tc_body(y_ref, p_hbm, out_ref, b0, b1, b2, b3, s0, s1, s2, s3):
    bufs = (b0, b1, b2, b3)
    sems = (s0, s1, s2, s3)

    def copy(i, slot):
        return pltpu.make_async_copy(
            p_hbm.at[pl.ds(i * _BR, _BR), :], bufs[slot], sems[slot]
        )

    for s in range(_NBUF):
        copy(s, s).start()

    col = jax.lax.broadcasted_iota(jnp.int32, (_BR, _C), 1)
    acc = jnp.float32(0.0)
    for i in range(_NCHUNK):
        slot = i % _NBUF
        copy(i, slot).wait()
        p = bufs[slot][...]
        yb = y_ref[pl.ds(i * _BR, _BR), :]
        acc += jnp.sum(p * p) - 2.0 * jnp.sum(jnp.where(col == yb, p, 0.0))
        if i + _NBUF < _NCHUNK:
            copy(i + _NBUF, slot).start()

    out_ref[0, 0] = acc


def _combine_body(tc_ref, sc_ref, out_ref):
    sc = sc_ref[...]
    total = tc_ref[0, 0] + jnp.sum(sc[:, 0:16]) - 2.0 * jnp.sum(sc[:, 16:32])
    out_ref[0, 0] = (total + jnp.float32(_B)) / jnp.float32(_B)


def kernel(probs, y):
    y32 = y.astype(jnp.int32)
    partials = _sc_partials(probs, y32)
    tc_part = pl.pallas_call(
        _tc_body,
        in_specs=[
            pl.BlockSpec(memory_space=pltpu.VMEM),
            pl.BlockSpec(memory_space=pl.ANY),
        ],
        out_specs=pl.BlockSpec(memory_space=pltpu.SMEM),
        out_shape=jax.ShapeDtypeStruct((1, 1), jnp.float32),
        scratch_shapes=(
            [pltpu.VMEM((_BR, _C), jnp.float32) for _ in range(_NBUF)]
            + [pltpu.SemaphoreType.DMA for _ in range(_NBUF)]
        ),
    )(y32[:_S].reshape(_S, 1), probs)
    out = pl.pallas_call(
        _combine_body,
        in_specs=[
            pl.BlockSpec(memory_space=pltpu.SMEM),
            pl.BlockSpec(memory_space=pltpu.VMEM),
        ],
        out_specs=pl.BlockSpec(memory_space=pltpu.SMEM),
        out_shape=jax.ShapeDtypeStruct((1, 1), jnp.float32),
    )(tc_part, partials)
    return out[0, 0]


# speedup vs baseline: 1.9175x; 1.2128x over previous
import functools

import jax
import jax.numpy as jnp
from jax import lax
from jax.experimental import pallas as pl
from jax.experimental.pallas import tpu as pltpu
from jax.experimental.pallas import tpu_sc as plsc

_B = 16384
_C = 1000
_S = 12800

_NW = 32
_RPW = (_B - _S) // _NW
_RB = 16
_NCH = _RPW // _RB

_BR = 1280
_NBUF = 4
_NCHUNK = _S // _BR


def _sc_body(p_hbm, y_hbm, out_hbm, bufA, bufB, yv, accv, sd0, sd1):
    w = lax.axis_index("s") * 2 + lax.axis_index("c")
    row0 = _S + w * _RPW
    sems = (sd0, sd1)

    bufs = (bufA, bufB)

    def copy(t, b):
        return pltpu.make_async_copy(
            p_hbm.at[pl.ds(row0 + t * _RB, _RB), :], bufs[b], sems[b]
        )

    copy(0, 0).start()
    pltpu.sync_copy(y_hbm.at[pl.ds(row0, _RPW)], yv)

    lane = lax.broadcasted_iota(jnp.int32, (16,), 0)
    zeros16 = jnp.zeros((16,), jnp.float32)

    a0 = zeros16
    a1 = zeros16
    gacc = zeros16
    for t in range(_NCH):
        b = t % 2
        if t + 1 < _NCH:
            copy(t + 1, 1 - b).start()
        copy(t, b).wait()

        for r in range(_RB):
            lr = t * _RB + r
            ybc = lax.gather(
                yv[pl.ds((lr // 16) * 16, 16)],
                jnp.full((16, 1), lr % 16, jnp.int32),
                lax.GatherDimensionNumbers(
                    offset_dims=(),
                    collapsed_slice_dims=(0,),
                    start_index_map=(0,),
                ),
                slice_sizes=(1,),
                mode=lax.GatherScatterMode.PROMISE_IN_BOUNDS,
            )

            def ibody(e, c, _b=b, _r=r, _ybc=ybc):
                a, g = c
                v = bufs[_b][_r, pl.ds(e, 16)]
                hit = (lane + e) == _ybc
                return a + v * v, g + jnp.where(hit, v, zeros16)

            a0, gacc = plsc.parallel_loop(
                0, 992, step=16, unroll=16, carry=(a0, gacc)
            )(ibody)
            vt = bufs[b][r, pl.ds(984, 16)]
            vt = jnp.where(lane >= 8, vt, zeros16)
            a0 = a0 + vt * vt
            gacc = gacc + jnp.where((lane + 984) == ybc, vt, zeros16)

    accv[pl.ds(0, 16)] = a0 + a1
    accv[pl.ds(16, 16)] = gacc
    pltpu.sync_copy(accv, out_hbm.at[w])


def _sc_partials(probs, y32):
    mesh = plsc.VectorSubcoreMesh(core_axis_name="c", subcore_axis_name="s")
    run = functools.partial(
        pl.kernel,
        mesh=mesh,
        out_type=jax.ShapeDtypeStruct((_NW, 32), jnp.float32),
        scratch_types=[
            pltpu.VMEM((_RB, _C), jnp.float32),
            pltpu.VMEM((_RB, _C), jnp.float32),
            pltpu.VMEM((_RPW,), jnp.int32),
            pltpu.VMEM((32,), jnp.float32),
            pltpu.SemaphoreType.DMA,
            pltpu.SemaphoreType.DMA,
        ],
    )(_sc_body)
    return run(probs, y32)


def _tc_body(y_ref, p_hbm, out_ref, b0, b1, b2, b3, s0, s1, s2, s3):
    bufs = (b0, b1, b2, b3)
    sems = (s0, s1, s2, s3)

    def copy(i, slot):
        return pltpu.make_async_copy(
            p_hbm.at[pl.ds(i * _BR, _BR), :], bufs[slot], sems[slot]
        )

    for s in range(_NBUF):
        copy(s, s).start()

    col = jax.lax.broadcasted_iota(jnp.int32, (_BR, _C), 1)
    acc = jnp.float32(0.0)
    for i in range(_NCHUNK):
        slot = i % _NBUF
        copy(i, slot).wait()
        p = bufs[slot][...]
        yb = y_ref[pl.ds(i * _BR, _BR), :]
        acc += jnp.sum(p * p) - 2.0 * jnp.sum(jnp.where(col == yb, p, 0.0))
        if i + _NBUF < _NCHUNK:
            copy(i + _NBUF, slot).start()

    out_ref[0, 0] = acc


def _combine_body(tc_ref, sc_ref, out_ref):
    sc = sc_ref[...]
    total = tc_ref[0, 0] + jnp.sum(sc[:, 0:16]) - 2.0 * jnp.sum(sc[:, 16:32])
    out_ref[0, 0] = (total + jnp.float32(_B)) / jnp.float32(_B)


def kernel(probs, y):
    y32 = y.astype(jnp.int32)
    partials = _sc_partials(probs, y32)
    tc_part = pl.pallas_call(
        _tc_body,
        in_specs=[
            pl.BlockSpec(memory_space=pltpu.VMEM),
            pl.BlockSpec(memory_space=pl.ANY),
        ],
        out_specs=pl.BlockSpec(memory_space=pltpu.SMEM),
        out_shape=jax.ShapeDtypeStruct((1, 1), jnp.float32),
        scratch_shapes=(
            [pltpu.VMEM((_BR, _C), jnp.float32) for _ in range(_NBUF)]
            + [pltpu.SemaphoreType.DMA for _ in range(_NBUF)]
        ),
    )(y32[:_S].reshape(_S, 1), probs)
    out = pl.pallas_call(
        _combine_body,
        in_specs=[
            pl.BlockSpec(memory_space=pltpu.SMEM),
            pl.BlockSpec(memory_space=pltpu.VMEM),
        ],
        out_specs=pl.BlockSpec(memory_space=pltpu.SMEM),
        out_shape=jax.ShapeDtypeStruct((1, 1), jnp.float32),
    )(tc_part, partials)
    return out[0, 0]
